# baseline (speedup 1.0000x reference)
import functools

import jax
import jax.numpy as jnp
from jax import lax
from jax.experimental import pallas as pl
from jax.experimental.pallas import tpu as pltpu
from jax.experimental.pallas import tpu_sc as plsc

LEVEL = 16
HIDDEN = 32
EMB_D = LEVEL * HIDDEN

NUM_CORES = 2
NUM_SUBCORES = 16
NW = NUM_CORES * NUM_SUBCORES

CHUNK = 80
NBUF = 2


def _make_gather(total, width, tc_tiling):
    assert total % NW == 0
    bpw = total // NW
    assert bpw % (CHUNK * NBUF) == 0
    nch = bpw // CHUNK
    ngroups = nch // NBUF

    mesh = plsc.VectorSubcoreMesh(core_axis_name="c", subcore_axis_name="s")

    @functools.partial(
        pl.kernel,
        mesh=mesh,
        compiler_params=pltpu.CompilerParams(use_tc_tiling_on_sc=tc_tiling),
        out_type=jax.ShapeDtypeStruct((total, width), jnp.float32),
        scratch_types=(
            [pltpu.VMEM((bpw,), jnp.int32)]
            + [pltpu.VMEM((CHUNK, width), jnp.float32) for _ in range(NBUF)]
            + [pltpu.SemaphoreType.DMA for _ in range(2 * NBUF)]
        ),
    )
    def gather_kernel(idx_hbm, table_hbm, out_hbm, idx_v, *scratch):
        bufs = scratch[:NBUF]
        gsem = scratch[NBUF:2 * NBUF]
        wsem = scratch[2 * NBUF:]

        wid = lax.axis_index("s") * NUM_CORES + lax.axis_index("c")
        base = wid * bpw
        pltpu.sync_copy(idx_hbm.at[pl.ds(pl.multiple_of(base, bpw), bpw)],
                        idx_v)

        def gather(j, b):
            off = pl.multiple_of(j * CHUNK, CHUNK)
            return pltpu.make_async_copy(
                table_hbm.at[idx_v.at[pl.ds(off, CHUNK)]], bufs[b], gsem[b])

        for b in range(NBUF):
            gather(b, b).start()

        def group(g, carry):
            for b in range(NBUF):
                j = g * NBUF + b
                row0 = pl.multiple_of(base + j * CHUNK, CHUNK)
                gather(j, b).wait()
                w = pltpu.make_async_copy(
                    bufs[b], out_hbm.at[pl.ds(row0, CHUNK), :], wsem[b])
                w.start()
                w.wait()

                @pl.when(g < ngroups - 1)
                def _():
                    gather(j + NBUF, b).start()
            return carry

        lax.fori_loop(0, ngroups, group, 0)

    return gather_kernel


BB = 256


def _tc_transpose(x):
    n, batch, w = x.shape

    def body(x_ref, o_ref):
        o_ref[...] = jnp.transpose(x_ref[...], (0, 2, 1))

    return pl.pallas_call(
        body,
        grid=(n, batch // BB),
        in_specs=[pl.BlockSpec((1, BB, w), lambda i, j: (i, j, 0))],
        out_specs=pl.BlockSpec((1, w, BB), lambda i, j: (i, 0, j)),
        out_shape=jax.ShapeDtypeStruct((n, w, batch), jnp.float32),
    )(x)


NCHUNKS = 5


def kernel(news_batch, news_id, news_repr_table, news_embedding_table):
    batch, nnews = news_id.shape
    total = batch * nnews
    idx = news_id.astype(jnp.int32).T.reshape(total)
    npc = nnews // NCHUNKS
    chunk = npc * batch
    emb_gather = _make_gather(chunk, EMB_D, tc_tiling=True)
    repr_gather = _make_gather(total, HIDDEN, tc_tiling=False)

    out_repr = repr_gather(idx, news_repr_table)
    gathered = [
        emb_gather(lax.dynamic_slice(idx, (c * chunk,), (chunk,)),
                   news_embedding_table)
        for c in range(NCHUNKS)
    ]
    repr_t = _tc_transpose(out_repr.reshape(nnews, batch, HIDDEN))
    emb_t = jnp.zeros((nnews, EMB_D, batch), jnp.float32)
    for c, g in enumerate(gathered):
        t = _tc_transpose(g.reshape(npc, batch, EMB_D))
        emb_t = lax.dynamic_update_slice(emb_t, t, (c * npc, 0, 0))
    news_embedding = emb_t.reshape(nnews, LEVEL, HIDDEN, batch).transpose(
        3, 0, 1, 2)
    news_repr = repr_t.transpose(2, 0, 1)
    return (news_embedding, news_repr)


# speedup vs baseline: 1.5925x; 1.5925x over previous
import functools

import jax
import jax.numpy as jnp
from jax import lax
from jax.experimental import pallas as pl
from jax.experimental.pallas import tpu as pltpu
from jax.experimental.pallas import tpu_sc as plsc

LEVEL = 16
HIDDEN = 32
EMB_D = LEVEL * HIDDEN

NUM_CORES = 2
NUM_SUBCORES = 16
NW = NUM_CORES * NUM_SUBCORES

CHUNK = 80
NBUF = 2


def _make_gather(total, width, tc_tiling):
    assert total % NW == 0
    bpw = total // NW
    assert bpw % (CHUNK * NBUF) == 0
    nch = bpw // CHUNK
    ngroups = nch // NBUF

    mesh = plsc.VectorSubcoreMesh(core_axis_name="c", subcore_axis_name="s")

    @functools.partial(
        pl.kernel,
        mesh=mesh,
        compiler_params=pltpu.CompilerParams(use_tc_tiling_on_sc=tc_tiling),
        out_type=jax.ShapeDtypeStruct((total, width), jnp.float32),
        scratch_types=(
            [pltpu.VMEM((bpw,), jnp.int32)]
            + [pltpu.VMEM((CHUNK, width), jnp.float32) for _ in range(NBUF)]
            + [pltpu.SemaphoreType.DMA for _ in range(2 * NBUF)]
        ),
    )
    def gather_kernel(idx_hbm, table_hbm, out_hbm, idx_v, *scratch):
        bufs = scratch[:NBUF]
        gsem = scratch[NBUF:2 * NBUF]
        wsem = scratch[2 * NBUF:]

        wid = lax.axis_index("s") * NUM_CORES + lax.axis_index("c")
        base = wid * bpw
        pltpu.sync_copy(idx_hbm.at[pl.ds(pl.multiple_of(base, bpw), bpw)],
                        idx_v)

        def gather(j, b):
            off = pl.multiple_of(j * CHUNK, CHUNK)
            return pltpu.make_async_copy(
                table_hbm.at[idx_v.at[pl.ds(off, CHUNK)]], bufs[b], gsem[b])

        for b in range(NBUF):
            gather(b, b).start()

        def group(g, carry):
            for b in range(NBUF):
                j = g * NBUF + b
                row0 = pl.multiple_of(base + j * CHUNK, CHUNK)
                gather(j, b).wait()
                w = pltpu.make_async_copy(
                    bufs[b], out_hbm.at[pl.ds(row0, CHUNK), :], wsem[b])
                w.start()
                w.wait()

                @pl.when(g < ngroups - 1)
                def _():
                    gather(j + NBUF, b).start()
            return carry

        lax.fori_loop(0, ngroups, group, 0)

    return gather_kernel


BB = 512


def _tc_transpose(x):
    n, batch, w = x.shape

    def body(x_ref, o_ref):
        o_ref[...] = jnp.transpose(x_ref[...], (0, 2, 1))

    return pl.pallas_call(
        body,
        grid=(n, batch // BB),
        in_specs=[pl.BlockSpec((1, BB, w), lambda i, j: (i, j, 0))],
        out_specs=pl.BlockSpec((1, w, BB), lambda i, j: (i, 0, j)),
        out_shape=jax.ShapeDtypeStruct((n, w, batch), jnp.float32),
    )(x)


def kernel(news_batch, news_id, news_repr_table, news_embedding_table):
    batch, nnews = news_id.shape
    total = batch * nnews
    idx = news_id.astype(jnp.int32).T.reshape(total)
    emb_gather = _make_gather(total, EMB_D, tc_tiling=True)
    repr_gather = _make_gather(total, HIDDEN, tc_tiling=False)
    out_emb = emb_gather(idx, news_embedding_table)
    out_repr = repr_gather(idx, news_repr_table)
    emb_t = _tc_transpose(out_emb.reshape(nnews, batch, EMB_D))
    repr_t = _tc_transpose(out_repr.reshape(nnews, batch, HIDDEN))
    news_embedding = emb_t.reshape(nnews, LEVEL, HIDDEN, batch).transpose(
        3, 0, 1, 2)
    news_repr = repr_t.transpose(2, 0, 1)
    return (news_embedding, news_repr)


# speedup vs baseline: 1.6326x; 1.0252x over previous
import functools

import jax
import jax.numpy as jnp
from jax import lax
from jax.experimental import pallas as pl
from jax.experimental.pallas import tpu as pltpu
from jax.experimental.pallas import tpu_sc as plsc

LEVEL = 16
HIDDEN = 32
EMB_D = LEVEL * HIDDEN

NUM_CORES = 2
NUM_SUBCORES = 16
NW = NUM_CORES * NUM_SUBCORES

CHUNK = 80
NBUF = 2


def _make_gather(total, width, tc_tiling):
    assert total % NW == 0
    bpw = total // NW
    assert bpw % (CHUNK * NBUF) == 0
    nch = bpw // CHUNK
    ngroups = nch // NBUF

    mesh = plsc.VectorSubcoreMesh(core_axis_name="c", subcore_axis_name="s")

    @functools.partial(
        pl.kernel,
        mesh=mesh,
        compiler_params=pltpu.CompilerParams(use_tc_tiling_on_sc=tc_tiling),
        out_type=jax.ShapeDtypeStruct((total, width), jnp.float32),
        scratch_types=(
            [pltpu.VMEM((bpw,), jnp.int32)]
            + [pltpu.VMEM((CHUNK, width), jnp.float32) for _ in range(NBUF)]
            + [pltpu.SemaphoreType.DMA for _ in range(2 * NBUF)]
        ),
    )
    def gather_kernel(idx_hbm, table_hbm, out_hbm, idx_v, *scratch):
        bufs = scratch[:NBUF]
        gsem = scratch[NBUF:2 * NBUF]
        wsem = scratch[2 * NBUF:]

        wid = lax.axis_index("s") * NUM_CORES + lax.axis_index("c")
        base = wid * bpw
        pltpu.sync_copy(idx_hbm.at[pl.ds(pl.multiple_of(base, bpw), bpw)],
                        idx_v)

        def gather(j, b):
            off = pl.multiple_of(j * CHUNK, CHUNK)
            return pltpu.make_async_copy(
                table_hbm.at[idx_v.at[pl.ds(off, CHUNK)]], bufs[b], gsem[b])

        for b in range(NBUF):
            gather(b, b).start()

        def group(g, carry):
            for b in range(NBUF):
                j = g * NBUF + b
                row0 = pl.multiple_of(base + j * CHUNK, CHUNK)
                gather(j, b).wait()
                w = pltpu.make_async_copy(
                    bufs[b], out_hbm.at[pl.ds(row0, CHUNK), :], wsem[b])
                w.start()
                w.wait()

                @pl.when(g < ngroups - 1)
                def _():
                    gather(j + NBUF, b).start()
            return carry

        lax.fori_loop(0, ngroups, group, 0)

    return gather_kernel


BB = 512


def _tc_transpose(x):
    n, batch, w = x.shape

    def body(x_ref, o_ref):
        o_ref[...] = jnp.transpose(x_ref[...], (0, 2, 1))

    return pl.pallas_call(
        body,
        grid=(n, batch // BB),
        in_specs=[pl.BlockSpec((1, BB, w), lambda i, j: (i, j, 0))],
        out_specs=pl.BlockSpec((1, w, BB), lambda i, j: (i, 0, j)),
        out_shape=jax.ShapeDtypeStruct((n, w, batch), jnp.float32),
    )(x)


def _tc_transpose_halves(x0, x1):
    nh, batch, w = x0.shape
    n = 2 * nh

    def body0(x_ref, o_ref):
        o_ref[...] = jnp.transpose(x_ref[...], (0, 2, 1))

    y = pl.pallas_call(
        body0,
        grid=(nh, batch // BB),
        in_specs=[pl.BlockSpec((1, BB, w), lambda i, j: (i, j, 0))],
        out_specs=pl.BlockSpec((1, w, BB), lambda i, j: (i, 0, j)),
        out_shape=jax.ShapeDtypeStruct((n, w, batch), jnp.float32),
    )(x0)

    def body1(full_ref, x_ref, o_ref):
        del full_ref
        o_ref[...] = jnp.transpose(x_ref[...], (0, 2, 1))

    return pl.pallas_call(
        body1,
        grid=(nh, batch // BB),
        in_specs=[
            pl.BlockSpec(memory_space=pl.ANY),
            pl.BlockSpec((1, BB, w), lambda i, j: (i, j, 0)),
        ],
        out_specs=pl.BlockSpec((1, w, BB), lambda i, j: (i + nh, 0, j)),
        out_shape=jax.ShapeDtypeStruct((n, w, batch), jnp.float32),
        input_output_aliases={0: 0},
    )(y, x1)


def kernel(news_batch, news_id, news_repr_table, news_embedding_table):
    batch, nnews = news_id.shape
    total = batch * nnews
    idx = news_id.astype(jnp.int32).T.reshape(total)
    half = total // 2
    nh = nnews // 2
    emb_gather = _make_gather(half, EMB_D, tc_tiling=True)
    repr_gather = _make_gather(total, HIDDEN, tc_tiling=False)
    out_emb0 = emb_gather(idx[:half], news_embedding_table)
    out_emb1 = emb_gather(idx[half:], news_embedding_table)
    out_repr = repr_gather(idx, news_repr_table)
    emb_t = _tc_transpose_halves(out_emb0.reshape(nh, batch, EMB_D),
                                 out_emb1.reshape(nh, batch, EMB_D))
    repr_t = _tc_transpose(out_repr.reshape(nnews, batch, HIDDEN))
    news_embedding = emb_t.reshape(nnews, LEVEL, HIDDEN, batch).transpose(
        3, 0, 1, 2)
    news_repr = repr_t.transpose(2, 0, 1)
    return (news_embedding, news_repr)
